# Initial kernel scaffold; baseline (speedup 1.0000x reference)
import functools

import jax
import jax.numpy as jnp
from jax import lax
from jax.experimental import pallas as pl
from jax.experimental.pallas import tpu as pltpu
from jax.experimental.pallas import tpu_sc as plsc

_BN = 256

_NC = 2
_NS = 16
_NW = _NC * _NS
_CH = 96


def _argmin_body(flat_ref, embed_ref, idx_ref, loss_ref, e2_ref):
    i = pl.program_id(0)
    nb = pl.num_programs(0)
    e = embed_ref[...]

    @pl.when(i == 0)
    def _():
        e2_ref[...] = jnp.sum(e * e, axis=1)
        loss_ref[0, 0] = 0.0

    f = flat_ref[...]
    mm = lax.dot_general(f, e, (((1,), (1,)), ((), ())),
                         preferred_element_type=jnp.float32)
    f2 = jnp.sum(f * f, axis=1, keepdims=True)
    dist = (f2 - 2.0 * mm) + e2_ref[...][None, :]
    m = jnp.min(dist, axis=1)
    k = dist.shape[1]
    cols = lax.broadcasted_iota(jnp.int32, dist.shape, 1)
    idx_ref[...] = jnp.min(jnp.where(dist == m[:, None], cols, k), axis=1)
    loss_ref[0, 0] += jnp.sum(m)

    @pl.when(i == nb - 1)
    def _():
        loss_ref[0, 0] = loss_ref[0, 0] * (0.25 / (nb * _BN * f.shape[1]))


def _argmin_call(flat, embed):
    n, d = flat.shape
    k = embed.shape[0]
    return pl.pallas_call(
        _argmin_body,
        grid=(n // _BN,),
        in_specs=[
            pl.BlockSpec((_BN, d), lambda i: (i, 0)),
            pl.BlockSpec((k, d), lambda i: (0, 0)),
        ],
        out_specs=[
            pl.BlockSpec((_BN,), lambda i: (i,)),
            pl.BlockSpec(block_shape=(1, 1), index_map=lambda i: (0, 0),
                         memory_space=pltpu.SMEM),
        ],
        out_shape=[
            jax.ShapeDtypeStruct((n,), jnp.int32),
            jax.ShapeDtypeStruct((1, 1), jnp.float32),
        ],
        scratch_shapes=[pltpu.VMEM((k,), jnp.float32)],
    )(flat, embed)


def _gather_call(embed, idx):
    n = idx.shape[0]
    d = embed.shape[1]
    bpw = n // _NW
    nch = bpw // _CH
    idx2 = idx.reshape(_NW * nch, _CH)
    mesh = plsc.VectorSubcoreMesh(core_axis_name="c", subcore_axis_name="s")

    @functools.partial(
        pl.kernel,
        out_type=jax.ShapeDtypeStruct((n, d), jnp.float32),
        mesh=mesh,
        scratch_types=[
            pltpu.VMEM((nch, _CH), jnp.int32),
            pltpu.VMEM((bpw, d), jnp.float32),
            pltpu.SemaphoreType.DMA,
        ],
    )
    def k(table_hbm, idx_hbm, out_hbm, idx_v, rows_v, sem):
        wid = lax.axis_index("s") * _NC + lax.axis_index("c")
        pltpu.sync_copy(idx_hbm.at[pl.ds(wid * nch, nch)], idx_v)
        copies = [
            pltpu.async_copy(table_hbm.at[idx_v.at[j]],
                             rows_v.at[pl.ds(j * _CH, _CH)], sem)
            for j in range(nch)
        ]
        for c in copies:
            c.wait()
        pltpu.sync_copy(rows_v, out_hbm.at[pl.ds(wid * bpw, bpw)])

    return k(embed, idx2)


def kernel(z, embed):
    b, t, d = z.shape
    n = b * t
    flat = z.reshape(n, d)
    idx, loss = _argmin_call(flat, embed)
    q = _gather_call(embed, idx)
    return q.reshape(b, t, d), idx.reshape(b, t), loss[0, 0]


# speedup vs baseline: 1.1569x; 1.1569x over previous
import functools

import jax
import jax.numpy as jnp
from jax import lax
from jax.experimental import pallas as pl
from jax.experimental.pallas import tpu as pltpu
from jax.experimental.pallas import tpu_sc as plsc

_BN = 256

_NC = 2
_NS = 16
_NW = _NC * _NS
_CH = 96


def _argmin_body(flat_ref, embed_ref, idx_ref, loss_ref, e2_ref):
    i = pl.program_id(0)
    nb = pl.num_programs(0)
    e = embed_ref[...]

    @pl.when(i == 0)
    def _():
        e2_ref[...] = jnp.sum(e * e, axis=1)
        loss_ref[0, 0] = 0.0

    f = flat_ref[...]
    mm = lax.dot_general(f, e, (((1,), (1,)), ((), ())),
                         preferred_element_type=jnp.float32)
    f2 = jnp.sum(f * f, axis=1, keepdims=True)
    dist = (f2 - 2.0 * mm) + e2_ref[...][None, :]
    m = jnp.min(dist, axis=1)
    k = dist.shape[1]
    cols = lax.broadcasted_iota(jnp.int32, dist.shape, 1)
    idx_ref[...] = jnp.min(jnp.where(dist == m[:, None], cols, k), axis=1)
    loss_ref[0, 0] += jnp.sum(m)

    @pl.when(i == nb - 1)
    def _():
        loss_ref[0, 0] = loss_ref[0, 0] * (0.25 / (nb * _BN * f.shape[1]))


def _argmin_call(flat, embed):
    n, d = flat.shape
    k = embed.shape[0]
    return pl.pallas_call(
        _argmin_body,
        grid=(n // _BN,),
        in_specs=[
            pl.BlockSpec((_BN, d), lambda i: (i, 0)),
            pl.BlockSpec((k, d), lambda i: (0, 0)),
        ],
        out_specs=[
            pl.BlockSpec((_BN,), lambda i: (i,)),
            pl.BlockSpec(block_shape=(1, 1), index_map=lambda i: (0, 0),
                         memory_space=pltpu.SMEM),
        ],
        out_shape=[
            jax.ShapeDtypeStruct((n,), jnp.int32),
            jax.ShapeDtypeStruct((1, 1), jnp.float32),
        ],
        scratch_shapes=[pltpu.VMEM((k,), jnp.float32)],
    )(flat, embed)


def _gather_call(embed, idx):
    n = idx.shape[0]
    d = embed.shape[1]
    bpw = n // _NW
    nch = bpw // _CH
    mesh = plsc.VectorSubcoreMesh(core_axis_name="c", subcore_axis_name="s")

    @functools.partial(
        pl.kernel,
        out_type=jax.ShapeDtypeStruct((n, d), jnp.float32),
        mesh=mesh,
        scratch_types=[
            pltpu.VMEM((bpw,), jnp.int32),
            pltpu.VMEM((bpw, d), jnp.float32),
            pltpu.SemaphoreType.DMA,
        ],
        compiler_params=pltpu.CompilerParams(use_tc_tiling_on_sc=False),
    )
    def k(table_hbm, idx_hbm, out_hbm, idx_v, rows_v, sem):
        wid = lax.axis_index("s") * _NC + lax.axis_index("c")
        pltpu.sync_copy(idx_hbm.at[pl.ds(wid * bpw, bpw)], idx_v)
        copies = [
            pltpu.async_copy(table_hbm.at[idx_v.at[pl.ds(j * _CH, _CH)]],
                             rows_v.at[pl.ds(j * _CH, _CH)], sem)
            for j in range(nch)
        ]
        for c in copies:
            c.wait()
        pltpu.sync_copy(rows_v, out_hbm.at[pl.ds(wid * bpw, bpw)])

    return k(embed, idx)


def kernel(z, embed):
    b, t, d = z.shape
    n = b * t
    flat = z.reshape(n, d)
    idx, loss = _argmin_call(flat, embed)
    q = _gather_call(embed, idx)
    return q.reshape(b, t, d), idx.reshape(b, t), loss[0, 0]
